# baseline (speedup 1.0000x reference)
import jax
import jax.numpy as jnp
from jax.experimental import pallas as pl
from jax.experimental.pallas import tpu as pltpu

NUM_SELECT = 300


def _prob_body(l_ref, c_ref, i_ref, o_ref):
    p = (
        (jax.nn.sigmoid(l_ref[...]) ** 0.45)
        * (jax.nn.sigmoid(c_ref[...]) ** 0.05)
        * (jax.nn.sigmoid(i_ref[...]) ** 0.5)
    )
    o_ref[...] = p


def kernel(pred_logits, pred_boxes, pred_centers, pred_ious, target_sizes, img_metas):
    B, N, C = pred_logits.shape
    total = B * N * C
    ROWS = total // 128
    BLK = 3640
    l2 = pred_logits.reshape(ROWS, 128)
    c2 = pred_centers.reshape(ROWS, 128)
    i2 = pred_ious.reshape(ROWS, 128)
    prob = pl.pallas_call(
        _prob_body,
        grid=(ROWS // BLK,),
        in_specs=[pl.BlockSpec((BLK, 128), lambda i: (i, 0))] * 3,
        out_specs=pl.BlockSpec((BLK, 128), lambda i: (i, 0)),
        out_shape=jax.ShapeDtypeStruct((ROWS, 128), jnp.float32),
    )(l2, c2, i2)
    flat = prob.reshape(B, N * C)
    topk_values, topk_indexes = jax.lax.top_k(flat, NUM_SELECT)
    scores = topk_values
    topk_boxes = topk_indexes // C
    labels = topk_indexes % C
    cx, cy, w, h = (pred_boxes[..., k] for k in range(4))
    boxes = jnp.stack([cx - 0.5 * w, cy - 0.5 * h, cx + 0.5 * w, cy + 0.5 * h], axis=-1)
    boxes = jnp.take_along_axis(boxes, topk_boxes[:, :, None], axis=1)
    img_h = target_sizes[:, 0].astype(jnp.float32)
    img_w = target_sizes[:, 1].astype(jnp.float32)
    scale_fct = jnp.stack([img_w, img_h, img_w, img_h], axis=1)
    boxes = boxes * scale_fct[:, None, :]
    return boxes, scores, labels
